# baseline (speedup 1.0000x reference)
import jax
import jax.numpy as jnp
from jax import lax
from jax.experimental import pallas as pl
from jax.experimental.pallas import tpu as pltpu
from jax.experimental.pallas import tpu_sc as plsc

B = 16384
S_DIM = 64
T_DIM = 32
R_DIM = S_DIM + T_DIM
CYCLE = 365

NC = 2
NS = 16
LANES = 16
NW = NC * NS
PER_W = B // NW
CHUNK = 64
NCHUNK = PER_W // CHUNK
NGROUP = CHUNK // LANES

_INV2PI = 0.15915494309189535
_MAGIC = 12582912.0
_C1 = 6.28125
_C2 = 1.9353071795864769e-3
_S0 = 0.9999782156662488
_S1 = -0.16662248279410358
_S2 = 0.008308176673817783
_S3 = -0.00019252550586158768
_S4 = 2.141589485971096e-06


def _psin(x):
    k = (x * _INV2PI + _MAGIC) - _MAGIC
    r = (x - k * _C1) - k * _C2
    t = r * r
    p = _S4 * t + _S3
    p = p * t + _S2
    p = p * t + _S1
    p = p * t + _S0
    return p * r


def _body(*refs):
    it = iter(refs)
    ent_h = next(it)
    ent_t = next(it)
    rel_f = next(it)
    rel_i = next(it)
    rtc = next(it)
    stw = next(it)
    time_tabs = [next(it) for _ in range(18)]
    heads = next(it)
    tails = next(it)
    rels = next(it)
    dates = next(it)
    datesd = next(it)
    yrf = next(it)
    mof = next(it)
    dyf = next(it)
    out = next(it)
    i_heads = next(it)
    i_tails = next(it)
    i_rels = next(it)
    i_dates = next(it)
    i_datesd = next(it)
    v_yr = next(it)
    v_mo = next(it)
    v_dy = next(it)
    b_h1s = next(it)
    b_t2s = next(it)
    b_h2s = next(it)
    b_t1s = next(it)
    b_rf = next(it)
    b_ri = next(it)
    b_tm = next(it)
    b_sw = next(it)
    b_time = [next(it) for _ in range(36)]
    v_score = next(it)
    sem = next(it)

    wid = lax.axis_index("s") * NC + lax.axis_index("c")

    def chunk_body(c, carry):
        base = wid * PER_W + c * CHUNK
        sl = pl.ds(base, CHUNK)
        pltpu.sync_copy(heads.at[sl], i_heads)
        pltpu.sync_copy(tails.at[sl], i_tails)
        pltpu.sync_copy(rels.at[sl], i_rels)
        pltpu.sync_copy(dates.at[sl], i_dates)
        pltpu.sync_copy(datesd.at[sl], i_datesd)
        pltpu.sync_copy(yrf.at[sl], v_yr)
        pltpu.sync_copy(mof.at[sl], v_mo)
        pltpu.sync_copy(dyf.at[sl], v_dy)

        cps = []

        def G(tbl, idxv, dst):
            cps.append(pltpu.async_copy(tbl.at[idxv], dst, sem))

        G(ent_h, i_heads, b_h1s)
        G(ent_t, i_heads, b_t2s)
        G(ent_h, i_tails, b_h2s)
        G(ent_t, i_tails, b_t1s)
        G(rel_f, i_rels, b_rf)
        G(rel_i, i_rels, b_ri)
        G(rtc, i_dates, b_tm)
        G(stw, i_datesd, b_sw)
        for srci, idxv in ((0, i_heads), (1, i_tails)):
            for j in range(18):
                G(time_tabs[j], idxv, b_time[srci * 18 + j])
        for cp in cps:
            cp.wait()

        def elem_body(el, carry2):
            eb = jnp.full((LANES,), el, jnp.int32)
            yr = plsc.load_gather(v_yr, [eb])
            mo = plsc.load_gather(v_mo, [eb])
            dy = plsc.load_gather(v_dy, [eb])

            acc = jnp.zeros((LANES,), jnp.float32)
            for v in range(S_DIM // LANES):
                cs = pl.ds(v * LANES, LANES)
                h1 = b_h1s[el, cs]
                t1 = b_t1s[el, cs]
                h2 = b_h2s[el, cs]
                t2 = b_t2s[el, cs]
                rf = b_rf[el, cs]
                ri = b_ri[el, cs]
                tm = b_tm[el, cs]
                r1 = rf + rf * tm
                r2 = ri + ri * tm
                acc = acc + h1 * r1 * t1 + h2 * r2 * t2

            for v in range(T_DIM // LANES):
                cs = pl.ds(v * LANES, LANES)
                cs96 = pl.ds(S_DIM + v * LANES, LANES)
                rf = b_rf[el, cs96]
                ri = b_ri[el, cs96]
                tm = b_tm[el, cs96]
                sw = b_sw[el, cs]

                def tte(srci, pfx):
                    tb = srci * 18 + pfx * 9
                    acc_t = None
                    for peri, tv in ((0, yr), (1, mo), (2, dy)):
                        fq = b_time[tb + 0 * 3 + peri][el, cs]
                        ph = b_time[tb + 1 * 3 + peri][el, cs]
                        am = b_time[tb + 2 * 3 + peri][el, cs]
                        term = am * _psin(fq * tv + ph)
                        acc_t = term if acc_t is None else acc_t + term
                    return acc_t + sw

                h1 = tte(0, 0)
                t1 = tte(1, 1)
                h2 = tte(1, 0)
                t2 = tte(0, 1)
                r1 = rf + rf * tm
                r2 = ri + ri * tm
                acc = acc + h1 * r1 * t1 + h2 * r2 * t2

            s = jnp.sum(acc) * 0.5
            lane = lax.iota(jnp.int32, 16)
            plsc.store_scatter(v_score, [eb], jnp.full((LANES,), s), mask=lane == 0)
            return carry2

        lax.fori_loop(0, CHUNK, elem_body, 0)
        pltpu.sync_copy(v_score, out.at[sl])
        return carry

    lax.fori_loop(0, NCHUNK, chunk_body, 0)


_SCRATCH = (
    [pltpu.VMEM((CHUNK,), jnp.int32)] * 5
    + [pltpu.VMEM((CHUNK,), jnp.float32)] * 3
    + [pltpu.VMEM((CHUNK, S_DIM), jnp.float32)] * 4
    + [pltpu.VMEM((CHUNK, R_DIM), jnp.float32)] * 3
    + [pltpu.VMEM((CHUNK, T_DIM), jnp.float32)]
    + [pltpu.VMEM((CHUNK, T_DIM), jnp.float32)] * 36
    + [pltpu.VMEM((CHUNK,), jnp.float32)]
    + [pltpu.SemaphoreType.DMA]
)

_sc_call = pl.kernel(
    _body,
    out_type=jax.ShapeDtypeStruct((B,), jnp.float32),
    mesh=plsc.VectorSubcoreMesh(core_axis_name="c", subcore_axis_name="s"),
    scratch_types=_SCRATCH,
    compiler_params=pltpu.CompilerParams(
        needs_layout_passes=False, use_tc_tiling_on_sc=False
    ),
)


def _transpose_body(src, dst):
    dst[...] = src[...].T


_TBLK = 2048


def _tc_transpose(arr):
    at = arr.T
    d, n = at.shape
    grid = ((n + _TBLK - 1) // _TBLK,)
    return pl.pallas_call(
        _transpose_body,
        grid=grid,
        in_specs=[pl.BlockSpec((d, _TBLK), lambda i: (0, i))],
        out_specs=pl.BlockSpec((_TBLK, d), lambda i: (i, 0)),
        out_shape=jax.ShapeDtypeStruct((n, d), jnp.float32),
    )(at)


def kernel(params, heads, rels, tails, years, months, days, date_ids):
    tables = [
        _tc_transpose(params["ent_embs_h"]),
        _tc_transpose(params["ent_embs_t"]),
        _tc_transpose(params["rel_embs_f"]),
        _tc_transpose(params["rel_embs_i"]),
        _tc_transpose(params["rtc"]),
        params["stw"],
    ]
    tables += [
        _tc_transpose(params[f"{per}_{kind}_{pfx}"])
        for pfx in ("h", "t")
        for kind in ("freq", "phi", "amps")
        for per in ("y", "m", "d")
    ]
    args = tables + [
        heads.astype(jnp.int32),
        tails.astype(jnp.int32),
        rels.astype(jnp.int32),
        date_ids.astype(jnp.int32),
        (date_ids // CYCLE).astype(jnp.int32),
        years.astype(jnp.float32),
        months.astype(jnp.float32),
        days.astype(jnp.float32),
    ]
    return _sc_call(*args)


# speedup vs baseline: 1.9077x; 1.9077x over previous
import jax
import jax.numpy as jnp
from jax import lax
from jax.experimental import pallas as pl
from jax.experimental.pallas import tpu as pltpu
from jax.experimental.pallas import tpu_sc as plsc

B = 16384
NUM_ENT = 100000
NUM_REL = 500
NUM_REL_PAD = 512
NUM_DATE = 4096
NUM_STW = NUM_DATE // 365 + 1
NUM_STW_PAD = 16
S_DIM = 64
T_DIM = 32
R_DIM = 96
CYCLE = 365
NFAC = 1728

NC = 2
NS = 16
LANES = 16
NW = NC * NS
CH = 8192
NB = B // CH

_INV2PI = 0.15915494309189535
_MAGIC = 12582912.0
_C1 = 6.28125
_C2 = 1.9353071795864769e-3
_S0 = 0.9999782156662488
_S1 = -0.16662248279410358
_S2 = 0.008308176673817783
_S3 = -0.00019252550586158768
_S4 = 2.141589485971096e-06


def _psin(x):
    k = (x * _INV2PI + _MAGIC) - _MAGIC
    r = (x - k * _C1) - k * _C2
    t = r * r
    p = _S4 * t + _S3
    p = p * t + _S2
    p = p * t + _S1
    p = p * t + _S0
    return p * r


def _stage_body(*refs):
    it = iter(refs)
    ent_h = next(it)
    ent_t = next(it)
    rel_f = next(it)
    rel_i = next(it)
    rtc = next(it)
    stw = next(it)
    time_tabs = [next(it) for _ in range(18)]
    heads = next(it)
    tails = next(it)
    rels = next(it)
    dates = next(it)
    datesd = next(it)
    staged = next(it)
    row_v = next(it)
    idx_v = next(it)
    fac_v = next(it)
    sem = next(it)

    wid = lax.axis_index("s") * NC + lax.axis_index("c")

    def gather_to(idx_hbm, fac_row):
        def bchunk(c, carry):
            b0 = c * CH
            pltpu.sync_copy(idx_hbm.at[pl.ds(b0, CH)], idx_v)

            @plsc.parallel_loop(0, CH // LANES, 1, unroll=4)
            def gl(i):
                sl = pl.ds(i * LANES, LANES)
                fac_v[sl] = plsc.load_gather(row_v, [idx_v[sl]])

            pltpu.sync_copy(fac_v, staged.at[fac_row, pl.ds(b0, CH)])
            return carry

        lax.fori_loop(0, NB, bchunk, 0)

    def do_table(tbl, n, d, row_base, sets):
        nj = (d + NW - 1) // NW

        def dim_loop(k, carry):
            j = k * NW + wid

            @pl.when(j < d)
            def _():
                pltpu.sync_copy(tbl.at[j], row_v.at[pl.ds(0, n)])
                for idx_hbm, set_off in sets:
                    gather_to(idx_hbm, row_base + set_off + j)

            return carry

        lax.fori_loop(0, nj, dim_loop, 0)

    do_table(ent_h, NUM_ENT, S_DIM, 0, [(heads, 0), (tails, 64)])
    do_table(ent_t, NUM_ENT, S_DIM, 128, [(heads, 0), (tails, 64)])
    do_table(rel_f, NUM_REL_PAD, R_DIM, 256, [(rels, 0)])
    do_table(rel_i, NUM_REL_PAD, R_DIM, 352, [(rels, 0)])
    do_table(rtc, NUM_DATE, R_DIM, 448, [(dates, 0)])
    do_table(stw, NUM_STW_PAD, T_DIM, 544, [(datesd, 0)])
    for t in range(18):
        do_table(
            time_tabs[t], NUM_ENT, T_DIM, 576 + t * T_DIM,
            [(heads, 0), (tails, 576)],
        )


_STAGE_SCRATCH = [
    pltpu.VMEM((NUM_ENT,), jnp.float32),
    pltpu.VMEM((CH,), jnp.int32),
    pltpu.VMEM((CH,), jnp.float32),
    pltpu.SemaphoreType.DMA,
]

_stage_call = pl.kernel(
    _stage_body,
    out_type=jax.ShapeDtypeStruct((NFAC, B), jnp.float32),
    mesh=plsc.VectorSubcoreMesh(core_axis_name="c", subcore_axis_name="s"),
    scratch_types=_STAGE_SCRATCH,
    compiler_params=pltpu.CompilerParams(
        needs_layout_passes=False, use_tc_tiling_on_sc=False
    ),
)

_CBLK = 1024


def _combine_body(st, yr_r, mo_r, dy_r, out_r):
    yr = yr_r[...]
    mo = mo_r[...]
    dy = dy_r[...]

    acc = jnp.zeros((_CBLK,), jnp.float32)
    for j in range(S_DIM):
        h1 = st[0 + j, :]
        h2 = st[64 + j, :]
        t2 = st[128 + j, :]
        t1 = st[192 + j, :]
        rf = st[256 + j, :]
        ri = st[352 + j, :]
        tm = st[448 + j, :]
        acc = acc + h1 * (rf + rf * tm) * t1 + h2 * (ri + ri * tm) * t2

    def tte(s, pfx, j):
        v = st[544 + j, :]
        for per, tv in ((0, yr), (1, mo), (2, dy)):
            fq = st[576 + s * 576 + (pfx * 9 + 0 * 3 + per) * T_DIM + j, :]
            ph = st[576 + s * 576 + (pfx * 9 + 1 * 3 + per) * T_DIM + j, :]
            am = st[576 + s * 576 + (pfx * 9 + 2 * 3 + per) * T_DIM + j, :]
            v = v + am * _psin(fq * tv + ph)
        return v

    for j in range(T_DIM):
        rf = st[256 + S_DIM + j, :]
        ri = st[352 + S_DIM + j, :]
        tm = st[448 + S_DIM + j, :]
        h1 = tte(0, 0, j)
        t1 = tte(1, 1, j)
        h2 = tte(1, 0, j)
        t2 = tte(0, 1, j)
        acc = acc + h1 * (rf + rf * tm) * t1 + h2 * (ri + ri * tm) * t2

    out_r[...] = acc * 0.5


def _combine_call(staged, yrf, mof, dyf):
    grid = (B // _CBLK,)
    return pl.pallas_call(
        _combine_body,
        grid=grid,
        in_specs=[
            pl.BlockSpec((NFAC, _CBLK), lambda i: (0, i)),
            pl.BlockSpec((_CBLK,), lambda i: (i,)),
            pl.BlockSpec((_CBLK,), lambda i: (i,)),
            pl.BlockSpec((_CBLK,), lambda i: (i,)),
        ],
        out_specs=pl.BlockSpec((_CBLK,), lambda i: (i,)),
        out_shape=jax.ShapeDtypeStruct((B,), jnp.float32),
    )(staged, yrf, mof, dyf)


def kernel(params, heads, rels, tails, years, months, days, date_ids):
    heads = heads.astype(jnp.int32)
    tails = tails.astype(jnp.int32)
    rels = rels.astype(jnp.int32)
    dates = date_ids.astype(jnp.int32)
    datesd = (date_ids // CYCLE).astype(jnp.int32)
    yrf = years.astype(jnp.float32)
    mof = months.astype(jnp.float32)
    dyf = days.astype(jnp.float32)

    time_tabs = [
        params[f"{per}_{kind}_{pfx}"].T
        for pfx in ("h", "t")
        for kind in ("freq", "phi", "amps")
        for per in ("y", "m", "d")
    ]
    rel_f_p = jnp.pad(params["rel_embs_f"], ((0, NUM_REL_PAD - NUM_REL), (0, 0)))
    rel_i_p = jnp.pad(params["rel_embs_i"], ((0, NUM_REL_PAD - NUM_REL), (0, 0)))
    stw_p = jnp.pad(params["stw"], ((0, NUM_STW_PAD - NUM_STW), (0, 0)))
    staged = _stage_call(
        params["ent_embs_h"].T,
        params["ent_embs_t"].T,
        rel_f_p.T,
        rel_i_p.T,
        params["rtc"].T,
        stw_p.T,
        *time_tabs,
        heads,
        tails,
        rels,
        dates,
        datesd,
    )
    return _combine_call(staged, yrf, mof, dyf)


# speedup vs baseline: 4.0778x; 2.1376x over previous
import jax
import jax.numpy as jnp
from jax import lax
from jax.experimental import pallas as pl
from jax.experimental.pallas import tpu as pltpu
from jax.experimental.pallas import tpu_sc as plsc

B = 16384
NUM_ENT = 100000
NUM_REL = 500
NUM_REL_PAD = 512
NUM_DATE = 4096
NUM_STW = NUM_DATE // 365 + 1
NUM_STW_PAD = 16
S_DIM = 64
T_DIM = 32
R_DIM = 96
CYCLE = 365
NFAC = 1728

NC = 2
NS = 16
LANES = 16
NW = NC * NS
CH = 8192
NB = B // CH

_INV2PI = 0.15915494309189535
_MAGIC = 12582912.0
_C1 = 6.28125
_C2 = 1.9353071795864769e-3
_S0 = 0.9999782156662488
_S1 = -0.16662248279410358
_S2 = 0.008308176673817783
_S3 = -0.00019252550586158768
_S4 = 2.141589485971096e-06


def _psin(x):
    k = (x * _INV2PI + _MAGIC) - _MAGIC
    r = (x - k * _C1) - k * _C2
    t = r * r
    p = _S4 * t + _S3
    p = p * t + _S2
    p = p * t + _S1
    p = p * t + _S0
    return p * r


def _mk_stage_helpers(wid, staged, row_v, idx_v, fac_v):
    def gather_to(idx_hbm, fac_row):
        def bchunk(c, carry):
            b0 = c * CH
            pltpu.sync_copy(idx_hbm.at[pl.ds(b0, CH)], idx_v)

            @plsc.parallel_loop(0, CH // LANES, 1, unroll=4)
            def gl(i):
                sl = pl.ds(i * LANES, LANES)
                fac_v[sl] = plsc.load_gather(row_v, [idx_v[sl]])

            pltpu.sync_copy(fac_v, staged.at[fac_row, pl.ds(b0, CH)])
            return carry

        lax.fori_loop(0, NB, bchunk, 0)

    def do_table(tbl, n, d, row_base, sets):
        nj = (d + NW - 1) // NW

        def dim_loop(k, carry):
            j = k * NW + wid

            @pl.when(j < d)
            def _():
                pltpu.sync_copy(tbl.at[j], row_v.at[pl.ds(0, n)])
                for idx_hbm, set_off in sets:
                    gather_to(idx_hbm, row_base + set_off + j)

            return carry

        lax.fori_loop(0, nj, dim_loop, 0)

    return do_table


def _stage_big_body(*refs):
    it = iter(refs)
    ent_h = next(it)
    ent_t = next(it)
    rel_f = next(it)
    rel_i = next(it)
    rtc = next(it)
    time_tabs = [next(it) for _ in range(18)]
    heads = next(it)
    tails = next(it)
    rels = next(it)
    dates = next(it)
    staged = next(it)
    row_v = next(it)
    idx_v = next(it)
    fac_v = next(it)
    sem = next(it)

    wid = lax.axis_index("s") * NC + lax.axis_index("c")
    do_table = _mk_stage_helpers(wid, staged, row_v, idx_v, fac_v)

    do_table(ent_h, NUM_ENT, S_DIM, 0, [(heads, 0), (tails, 64)])
    do_table(ent_t, NUM_ENT, S_DIM, 128, [(heads, 0), (tails, 64)])
    do_table(rel_f, NUM_REL_PAD, R_DIM, 256, [(rels, 0)])
    do_table(rel_i, NUM_REL_PAD, R_DIM, 352, [(rels, 0)])
    do_table(rtc, NUM_DATE, R_DIM, 448, [(dates, 0)])
    for t in range(18):
        do_table(
            time_tabs[t], NUM_ENT, T_DIM, 544 + t * T_DIM,
            [(heads, 0), (tails, 576)],
        )


def _stage_small_body(*refs):
    it = iter(refs)
    stw = next(it)
    datesd = next(it)
    staged = next(it)
    row_v = next(it)
    idx_v = next(it)
    fac_v = next(it)
    sem = next(it)

    wid = lax.axis_index("s") * NC + lax.axis_index("c")
    do_table = _mk_stage_helpers(wid, staged, row_v, idx_v, fac_v)

    do_table(stw, NUM_STW_PAD, T_DIM, 0, [(datesd, 0)])


NBIG = 256 + 96 * 3 + 18 * T_DIM * 2
NSMALL = T_DIM

_STAGE_SCRATCH = [
    pltpu.VMEM((NUM_ENT,), jnp.float32),
    pltpu.VMEM((CH,), jnp.int32),
    pltpu.VMEM((CH,), jnp.float32),
    pltpu.SemaphoreType.DMA,
]

_SC_MESH = plsc.VectorSubcoreMesh(core_axis_name="c", subcore_axis_name="s")

_stage_big_call = pl.kernel(
    _stage_big_body,
    out_type=jax.ShapeDtypeStruct((NBIG, B), jnp.float32),
    mesh=_SC_MESH,
    scratch_types=_STAGE_SCRATCH,
    compiler_params=pltpu.CompilerParams(
        needs_layout_passes=False, use_tc_tiling_on_sc=True
    ),
)

_stage_small_call = pl.kernel(
    _stage_small_body,
    out_type=jax.ShapeDtypeStruct((NSMALL, B), jnp.float32),
    mesh=_SC_MESH,
    scratch_types=_STAGE_SCRATCH,
    compiler_params=pltpu.CompilerParams(
        needs_layout_passes=False, use_tc_tiling_on_sc=False
    ),
)

_CBLK = 1024


def _combine_body(stb, sts, yr_r, mo_r, dy_r, out_r):
    yr = yr_r[...]
    mo = mo_r[...]
    dy = dy_r[...]

    acc = jnp.zeros((_CBLK,), jnp.float32)
    for j in range(S_DIM):
        h1 = stb[0 + j, :]
        h2 = stb[64 + j, :]
        t2 = stb[128 + j, :]
        t1 = stb[192 + j, :]
        rf = stb[256 + j, :]
        ri = stb[352 + j, :]
        tm = stb[448 + j, :]
        acc = acc + h1 * (rf + rf * tm) * t1 + h2 * (ri + ri * tm) * t2

    def tte(s, pfx, j):
        v = sts[j, :]
        for per, tv in ((0, yr), (1, mo), (2, dy)):
            fq = stb[544 + s * 576 + (pfx * 9 + 0 * 3 + per) * T_DIM + j, :]
            ph = stb[544 + s * 576 + (pfx * 9 + 1 * 3 + per) * T_DIM + j, :]
            am = stb[544 + s * 576 + (pfx * 9 + 2 * 3 + per) * T_DIM + j, :]
            v = v + am * _psin(fq * tv + ph)
        return v

    for j in range(T_DIM):
        rf = stb[256 + S_DIM + j, :]
        ri = stb[352 + S_DIM + j, :]
        tm = stb[448 + S_DIM + j, :]
        h1 = tte(0, 0, j)
        t1 = tte(1, 1, j)
        h2 = tte(1, 0, j)
        t2 = tte(0, 1, j)
        acc = acc + h1 * (rf + rf * tm) * t1 + h2 * (ri + ri * tm) * t2

    out_r[...] = acc * 0.5


def _combine_call(staged_big, staged_small, yrf, mof, dyf):
    grid = (B // _CBLK,)
    vec = pl.BlockSpec((_CBLK,), lambda i: (i,))
    return pl.pallas_call(
        _combine_body,
        grid=grid,
        in_specs=[
            pl.BlockSpec((NBIG, _CBLK), lambda i: (0, i)),
            pl.BlockSpec((NSMALL, _CBLK), lambda i: (0, i)),
            vec,
            vec,
            vec,
        ],
        out_specs=vec,
        out_shape=jax.ShapeDtypeStruct((B,), jnp.float32),
    )(staged_big, staged_small, yrf, mof, dyf)


def kernel(params, heads, rels, tails, years, months, days, date_ids):
    heads = heads.astype(jnp.int32)
    tails = tails.astype(jnp.int32)
    rels = rels.astype(jnp.int32)
    dates = date_ids.astype(jnp.int32)
    datesd = (date_ids // CYCLE).astype(jnp.int32)
    yrf = years.astype(jnp.float32)
    mof = months.astype(jnp.float32)
    dyf = days.astype(jnp.float32)

    time_tabs = [
        params[f"{per}_{kind}_{pfx}"].T
        for pfx in ("h", "t")
        for kind in ("freq", "phi", "amps")
        for per in ("y", "m", "d")
    ]
    rel_f_p = jnp.pad(params["rel_embs_f"], ((0, NUM_REL_PAD - NUM_REL), (0, 0)))
    rel_i_p = jnp.pad(params["rel_embs_i"], ((0, NUM_REL_PAD - NUM_REL), (0, 0)))
    stw_p = jnp.pad(params["stw"], ((0, NUM_STW_PAD - NUM_STW), (0, 0)))
    staged_small = _stage_small_call(
        stw_p.T,
        datesd,
    )
    staged_big = _stage_big_call(
        params["ent_embs_h"].T,
        params["ent_embs_t"].T,
        rel_f_p.T,
        rel_i_p.T,
        params["rtc"].T,
        *time_tabs,
        heads,
        tails,
        rels,
        dates,
    )
    return _combine_call(staged_big, staged_small, yrf, mof, dyf)


# speedup vs baseline: 4.9523x; 1.2144x over previous
import jax
import jax.numpy as jnp
from jax import lax
from jax.experimental import pallas as pl
from jax.experimental.pallas import tpu as pltpu
from jax.experimental.pallas import tpu_sc as plsc

B = 16384
NUM_ENT = 100000
NUM_REL = 500
NUM_REL_PAD = 512
NUM_DATE = 4096
NUM_STW = NUM_DATE // 365 + 1
NUM_STW_PAD = 16
S_DIM = 64
T_DIM = 32
R_DIM = 96
CYCLE = 365
NFAC = 1728

NC = 2
NS = 16
LANES = 16
NW = NC * NS
CH = 8192
NB = B // CH

_INV2PI = 0.15915494309189535
_MAGIC = 12582912.0
_C1 = 6.28125
_C2 = 1.9353071795864769e-3
_S0 = 0.9999782156662488
_S1 = -0.16662248279410358
_S2 = 0.008308176673817783
_S3 = -0.00019252550586158768
_S4 = 2.141589485971096e-06


def _psin(x):
    k = (x * _INV2PI + _MAGIC) - _MAGIC
    r = (x - k * _C1) - k * _C2
    t = r * r
    p = _S4 * t + _S3
    p = p * t + _S2
    p = p * t + _S1
    p = p * t + _S0
    return p * r


def _mk_stage_helpers(wid, staged, row_v, idx_bufs, fac_v, sem_row, sem_idx):
    idx_sems = (sem_idx[0], sem_idx[1])
    def do_table(tbl, n, d, row_base, sets):
        nj = (d + NW - 1) // NW
        subs = [
            (idx_hbm, c, set_off)
            for idx_hbm, set_off in sets
            for c in range(NB)
        ]

        def dim_loop(k, carry):
            j = k * NW + wid

            @pl.when(j < d)
            def _():
                rcp = pltpu.async_copy(tbl.at[j], row_v.at[pl.ds(0, n)], sem_row)
                ih0, c0, _ = subs[0]
                icp = pltpu.async_copy(
                    ih0.at[pl.ds(c0 * CH, CH)], idx_bufs[0], idx_sems[0]
                )
                rcp.wait()
                for si, (idx_hbm, c, set_off) in enumerate(subs):
                    icp.wait()
                    if si + 1 < len(subs):
                        nih, nc, _ = subs[si + 1]
                        icp = pltpu.async_copy(
                            nih.at[pl.ds(nc * CH, CH)],
                            idx_bufs[(si + 1) % 2],
                            idx_sems[(si + 1) % 2],
                        )
                    ib = idx_bufs[si % 2]

                    @plsc.parallel_loop(0, CH // LANES, 1, unroll=4)
                    def gl(i):
                        sl = pl.ds(i * LANES, LANES)
                        fac_v[sl] = plsc.load_gather(row_v, [ib[sl]])

                    pltpu.sync_copy(
                        fac_v, staged.at[row_base + set_off + j, pl.ds(c * CH, CH)]
                    )

            return carry

        lax.fori_loop(0, nj, dim_loop, 0)

    return do_table


def _stage_big_body(*refs):
    it = iter(refs)
    ent_h = next(it)
    ent_t = next(it)
    rel_f = next(it)
    rel_i = next(it)
    rtc = next(it)
    time_tabs = [next(it) for _ in range(18)]
    heads = next(it)
    tails = next(it)
    rels = next(it)
    dates = next(it)
    staged = next(it)
    row_v = next(it)
    idx_b0 = next(it)
    idx_b1 = next(it)
    fac_v = next(it)
    sem_row = next(it)
    sem_i0 = next(it)
    sem_i1 = next(it)

    wid = lax.axis_index("s") * NC + lax.axis_index("c")
    do_table = _mk_stage_helpers(
        wid, staged, row_v, [idx_b0, idx_b1], fac_v, sem_row, (sem_i0, sem_i1)
    )

    do_table(ent_h, NUM_ENT, S_DIM, 0, [(heads, 0), (tails, 64)])
    do_table(ent_t, NUM_ENT, S_DIM, 128, [(heads, 0), (tails, 64)])
    do_table(rel_f, NUM_REL_PAD, R_DIM, 256, [(rels, 0)])
    do_table(rel_i, NUM_REL_PAD, R_DIM, 352, [(rels, 0)])
    do_table(rtc, NUM_DATE, R_DIM, 448, [(dates, 0)])
    for t in range(18):
        do_table(
            time_tabs[t], NUM_ENT, T_DIM, 544 + t * T_DIM,
            [(heads, 0), (tails, 576)],
        )


def _stage_small_body(*refs):
    it = iter(refs)
    stw = next(it)
    datesd = next(it)
    staged = next(it)
    row_v = next(it)
    idx_b0 = next(it)
    idx_b1 = next(it)
    fac_v = next(it)
    sem_row = next(it)
    sem_i0 = next(it)
    sem_i1 = next(it)

    wid = lax.axis_index("s") * NC + lax.axis_index("c")
    do_table = _mk_stage_helpers(
        wid, staged, row_v, [idx_b0, idx_b1], fac_v, sem_row, (sem_i0, sem_i1)
    )

    do_table(stw, NUM_STW_PAD, T_DIM, 0, [(datesd, 0)])


NBIG = 256 + 96 * 3 + 18 * T_DIM * 2
NSMALL = T_DIM

_STAGE_SCRATCH = [
    pltpu.VMEM((NUM_ENT,), jnp.float32),
    pltpu.VMEM((CH,), jnp.int32),
    pltpu.VMEM((CH,), jnp.int32),
    pltpu.VMEM((CH,), jnp.float32),
    pltpu.SemaphoreType.DMA,
    pltpu.SemaphoreType.DMA,
    pltpu.SemaphoreType.DMA,
]

_SC_MESH = plsc.VectorSubcoreMesh(core_axis_name="c", subcore_axis_name="s")

_stage_big_call = pl.kernel(
    _stage_big_body,
    out_type=jax.ShapeDtypeStruct((NBIG, B), jnp.float32),
    mesh=_SC_MESH,
    scratch_types=_STAGE_SCRATCH,
    compiler_params=pltpu.CompilerParams(
        needs_layout_passes=False, use_tc_tiling_on_sc=True
    ),
)

_stage_small_call = pl.kernel(
    _stage_small_body,
    out_type=jax.ShapeDtypeStruct((NSMALL, B), jnp.float32),
    mesh=_SC_MESH,
    scratch_types=_STAGE_SCRATCH,
    compiler_params=pltpu.CompilerParams(
        needs_layout_passes=False, use_tc_tiling_on_sc=False
    ),
)

_CBLK = 1024


def _combine_body(stb, sts, yr_r, mo_r, dy_r, out_r):
    yr = yr_r[...]
    mo = mo_r[...]
    dy = dy_r[...]

    acc = jnp.zeros((_CBLK,), jnp.float32)
    for j in range(S_DIM):
        h1 = stb[0 + j, :]
        h2 = stb[64 + j, :]
        t2 = stb[128 + j, :]
        t1 = stb[192 + j, :]
        rf = stb[256 + j, :]
        ri = stb[352 + j, :]
        tm = stb[448 + j, :]
        acc = acc + h1 * (rf + rf * tm) * t1 + h2 * (ri + ri * tm) * t2

    def tte(s, pfx, j):
        v = sts[j, :]
        for per, tv in ((0, yr), (1, mo), (2, dy)):
            fq = stb[544 + s * 576 + (pfx * 9 + 0 * 3 + per) * T_DIM + j, :]
            ph = stb[544 + s * 576 + (pfx * 9 + 1 * 3 + per) * T_DIM + j, :]
            am = stb[544 + s * 576 + (pfx * 9 + 2 * 3 + per) * T_DIM + j, :]
            v = v + am * _psin(fq * tv + ph)
        return v

    for j in range(T_DIM):
        rf = stb[256 + S_DIM + j, :]
        ri = stb[352 + S_DIM + j, :]
        tm = stb[448 + S_DIM + j, :]
        h1 = tte(0, 0, j)
        t1 = tte(1, 1, j)
        h2 = tte(1, 0, j)
        t2 = tte(0, 1, j)
        acc = acc + h1 * (rf + rf * tm) * t1 + h2 * (ri + ri * tm) * t2

    out_r[...] = acc * 0.5


def _combine_call(staged_big, staged_small, yrf, mof, dyf):
    grid = (B // _CBLK,)
    vec = pl.BlockSpec((_CBLK,), lambda i: (i,))
    return pl.pallas_call(
        _combine_body,
        grid=grid,
        in_specs=[
            pl.BlockSpec((NBIG, _CBLK), lambda i: (0, i)),
            pl.BlockSpec((NSMALL, _CBLK), lambda i: (0, i)),
            vec,
            vec,
            vec,
        ],
        out_specs=vec,
        out_shape=jax.ShapeDtypeStruct((B,), jnp.float32),
    )(staged_big, staged_small, yrf, mof, dyf)


def kernel(params, heads, rels, tails, years, months, days, date_ids):
    heads = heads.astype(jnp.int32)
    tails = tails.astype(jnp.int32)
    rels = rels.astype(jnp.int32)
    dates = date_ids.astype(jnp.int32)
    datesd = (date_ids // CYCLE).astype(jnp.int32)
    yrf = years.astype(jnp.float32)
    mof = months.astype(jnp.float32)
    dyf = days.astype(jnp.float32)

    time_tabs = [
        params[f"{per}_{kind}_{pfx}"].T
        for pfx in ("h", "t")
        for kind in ("freq", "phi", "amps")
        for per in ("y", "m", "d")
    ]
    rel_f_p = jnp.pad(params["rel_embs_f"], ((0, NUM_REL_PAD - NUM_REL), (0, 0)))
    rel_i_p = jnp.pad(params["rel_embs_i"], ((0, NUM_REL_PAD - NUM_REL), (0, 0)))
    stw_p = jnp.pad(params["stw"], ((0, NUM_STW_PAD - NUM_STW), (0, 0)))
    staged_small = _stage_small_call(
        stw_p.T,
        datesd,
    )
    staged_big = _stage_big_call(
        params["ent_embs_h"].T,
        params["ent_embs_t"].T,
        rel_f_p.T,
        rel_i_p.T,
        params["rtc"].T,
        *time_tabs,
        heads,
        tails,
        rels,
        dates,
    )
    return _combine_call(staged_big, staged_small, yrf, mof, dyf)
